# baseline (device time: 14009 ns/iter reference)
import jax
import jax.numpy as jnp
from jax import lax
from jax.experimental import pallas as pl
from jax.experimental.pallas import tpu as pltpu

N_DEV = 8
EXPERTS_PER_DEV = 2
HALVES = 2


def kernel(x, router_W, route_idx, expert_W):
    del router_W
    n, d = x.shape
    h = expert_W.shape[-1]
    B = n // N_DEV
    hw = h // HALVES

    def body(x_ref, idx_ref, ew_ref, out_ref, acc_ref, abf_ref, a2a_ref,
             cstage_ref, b_ssems, b_rsems, c_ssems, c_rsems):
        my = lax.axis_index("i")

        barrier_sem = pltpu.get_barrier_semaphore()
        for j in range(1, N_DEV):
            pl.semaphore_signal(
                barrier_sem, inc=1,
                device_id=((my + j) % N_DEV,),
                device_id_type=pl.DeviceIdType.MESH,
            )

        idx = idx_ref[:, :]
        xb = x_ref[:, :].astype(jnp.bfloat16)
        acc = jnp.zeros((n, h), jnp.float32)
        for e in range(EXPERTS_PER_DEV):
            ge = my * EXPERTS_PER_DEV + e
            y = jnp.dot(xb, ew_ref[e, :, :].astype(jnp.bfloat16),
                        preferred_element_type=jnp.float32)
            acc = acc + jnp.where(idx == ge, y, 0.0)
        acc_ref[:, :] = acc
        for f in range(HALVES):
            abf_ref[f] = acc[:, f * hw:(f + 1) * hw].astype(jnp.bfloat16)

        pl.semaphore_wait(barrier_sem, N_DEV - 1)

        bdescs = {}
        for f in range(HALVES):
            for j in range(1, N_DEV):
                r = (my + j) % N_DEV
                dsc = pltpu.make_async_remote_copy(
                    src_ref=abf_ref.at[f, pl.ds(r * B, B), :],
                    dst_ref=a2a_ref.at[f, j - 1],
                    send_sem=b_ssems.at[f, j - 1],
                    recv_sem=b_rsems.at[f, j - 1],
                    device_id=(r,),
                    device_id_type=pl.DeviceIdType.MESH,
                )
                dsc.start()
                bdescs[(f, j)] = dsc

        cdescs = {}
        blks = []
        for f in range(HALVES):
            blk = acc_ref[pl.ds(my * B, B), f * hw:(f + 1) * hw]
            for j in range(1, N_DEV):
                bdescs[(f, j)].wait()
                blk = blk + a2a_ref[f, j - 1].astype(jnp.float32)
            blks.append(blk)
            cstage_ref[f, my] = blk.astype(jnp.bfloat16)
            for j in range(1, N_DEV):
                r = (my + j) % N_DEV
                dsc = pltpu.make_async_remote_copy(
                    src_ref=cstage_ref.at[f, my],
                    dst_ref=cstage_ref.at[f, my],
                    send_sem=c_ssems.at[f, j - 1],
                    recv_sem=c_rsems.at[f, j - 1],
                    device_id=(r,),
                    device_id_type=pl.DeviceIdType.MESH,
                )
                dsc.start()
                cdescs[(f, j)] = dsc

        for f in range(HALVES):
            for j in range(1, N_DEV):
                cdescs[(f, j)].wait()
            out_ref[:, f * hw:(f + 1) * hw] = (
                cstage_ref[f].reshape(n, hw).astype(jnp.float32))
        for f in range(HALVES):
            out_ref[pl.ds(my * B, B), f * hw:(f + 1) * hw] = blks[f]

    return pl.pallas_call(
        body,
        out_shape=jax.ShapeDtypeStruct((n, h), jnp.float32),
        in_specs=[
            pl.BlockSpec(memory_space=pltpu.VMEM),
            pl.BlockSpec(memory_space=pltpu.VMEM),
            pl.BlockSpec(memory_space=pltpu.VMEM),
        ],
        out_specs=pl.BlockSpec(memory_space=pltpu.VMEM),
        scratch_shapes=[
            pltpu.VMEM((n, h), jnp.float32),
            pltpu.VMEM((HALVES, n, hw), jnp.bfloat16),
            pltpu.VMEM((HALVES, N_DEV - 1, B, hw), jnp.bfloat16),
            pltpu.VMEM((HALVES, N_DEV, B, hw), jnp.bfloat16),
            pltpu.SemaphoreType.DMA((HALVES, N_DEV - 1)),
            pltpu.SemaphoreType.DMA((HALVES, N_DEV - 1)),
            pltpu.SemaphoreType.DMA((HALVES, N_DEV - 1)),
            pltpu.SemaphoreType.DMA((HALVES, N_DEV - 1)),
        ],
        compiler_params=pltpu.CompilerParams(collective_id=0),
    )(x, route_idx, expert_W)


# device time: 13983 ns/iter; 1.0019x vs baseline; 1.0019x over previous
import jax
import jax.numpy as jnp
from jax import lax
from jax.experimental import pallas as pl
from jax.experimental.pallas import tpu as pltpu

N_DEV = 8
EXPERTS_PER_DEV = 2


def kernel(x, router_W, route_idx, expert_W):
    del router_W
    n, d = x.shape
    h = expert_W.shape[-1]
    B = n // N_DEV

    def body(x_ref, idx_ref, ew_ref, out_ref, acc_ref, abf_ref, a2a_ref,
             cstage_ref, b_ssems, b_rsems, c_ssems, c_rsems):
        my = lax.axis_index("i")

        barrier_sem = pltpu.get_barrier_semaphore()
        for j in range(1, N_DEV):
            pl.semaphore_signal(
                barrier_sem, inc=1,
                device_id=((my + j) % N_DEV,),
                device_id_type=pl.DeviceIdType.MESH,
            )

        idx = idx_ref[:, :]
        xb = x_ref[:, :].astype(jnp.bfloat16)
        acc = jnp.zeros((n, h), jnp.float32)
        for e in range(EXPERTS_PER_DEV):
            ge = my * EXPERTS_PER_DEV + e
            y = jnp.dot(xb, ew_ref[e, :, :].astype(jnp.bfloat16),
                        preferred_element_type=jnp.float32)
            acc = acc + jnp.where(idx == ge, y, 0.0)
        acc_ref[:, :] = acc
        abf_ref[:, :] = acc.astype(jnp.bfloat16)

        pl.semaphore_wait(barrier_sem, N_DEV - 1)

        bdescs = []
        for j in range(1, N_DEV):
            r = (my + j) % N_DEV
            dsc = pltpu.make_async_remote_copy(
                src_ref=abf_ref.at[pl.ds(r * B, B), :],
                dst_ref=a2a_ref.at[j - 1],
                send_sem=b_ssems.at[j - 1],
                recv_sem=b_rsems.at[j - 1],
                device_id=(r,),
                device_id_type=pl.DeviceIdType.MESH,
            )
            dsc.start()
            bdescs.append(dsc)
        for dsc in bdescs:
            dsc.wait()

        blk = acc_ref[pl.ds(my * B, B), :]
        for j in range(N_DEV - 1):
            blk = blk + a2a_ref[j].astype(jnp.float32)
        cstage_ref[my] = blk.astype(jnp.bfloat16)

        cdescs = []
        for j in range(1, N_DEV):
            r = (my + j) % N_DEV
            dsc = pltpu.make_async_remote_copy(
                src_ref=cstage_ref.at[my],
                dst_ref=cstage_ref.at[my],
                send_sem=c_ssems.at[j - 1],
                recv_sem=c_rsems.at[j - 1],
                device_id=(r,),
                device_id_type=pl.DeviceIdType.MESH,
            )
            dsc.start()
            cdescs.append(dsc)
        for dsc in cdescs:
            dsc.wait()

        out_ref[:, :] = cstage_ref[:, :, :].reshape(n, h).astype(jnp.float32)
        out_ref[pl.ds(my * B, B), :] = blk

    return pl.pallas_call(
        body,
        out_shape=jax.ShapeDtypeStruct((n, h), jnp.float32),
        in_specs=[
            pl.BlockSpec(memory_space=pltpu.VMEM),
            pl.BlockSpec(memory_space=pltpu.VMEM),
            pl.BlockSpec(memory_space=pltpu.VMEM),
        ],
        out_specs=pl.BlockSpec(memory_space=pltpu.VMEM),
        scratch_shapes=[
            pltpu.VMEM((n, h), jnp.float32),
            pltpu.VMEM((n, h), jnp.bfloat16),
            pltpu.VMEM((N_DEV - 1, B, h), jnp.bfloat16),
            pltpu.VMEM((N_DEV, B, h), jnp.bfloat16),
            pltpu.SemaphoreType.DMA((N_DEV - 1,)),
            pltpu.SemaphoreType.DMA((N_DEV - 1,)),
            pltpu.SemaphoreType.DMA((N_DEV - 1,)),
            pltpu.SemaphoreType.DMA((N_DEV - 1,)),
        ],
        compiler_params=pltpu.CompilerParams(collective_id=0),
    )(x, route_idx, expert_W)


# device time: 10750 ns/iter; 1.3032x vs baseline; 1.3007x over previous
import jax
import jax.numpy as jnp
from jax import lax
from jax.experimental import pallas as pl
from jax.experimental.pallas import tpu as pltpu

N_DEV = 8
EXPERTS_PER_DEV = 2
CAP = 64


def kernel(x, router_W, route_idx, expert_W):
    del router_W
    n, d = x.shape
    h = expert_W.shape[-1]

    def body(x_ref, idx_ref, ew_ref, out_ref, stage_ref, u_ref,
             ssems, rsems):
        my = lax.axis_index("i")

        barrier_sem = pltpu.get_barrier_semaphore()
        for j in range(1, N_DEV):
            pl.semaphore_signal(
                barrier_sem, inc=1,
                device_id=((my + j) % N_DEV,),
                device_id_type=pl.DeviceIdType.MESH,
            )

        idx = idx_ref[:, :]
        dev = idx // EXPERTS_PER_DEV

        rows_i = lax.broadcasted_iota(jnp.int32, (n, n), 0)
        cols_j = lax.broadcasted_iota(jnp.int32, (n, n), 1)
        ltri = jnp.where(cols_j < rows_i, 1.0, 0.0)
        kio = lax.broadcasted_iota(jnp.int32, (n, CAP), 1)
        m_my = dev == my
        rank_my = jnp.dot(ltri, m_my.astype(jnp.float32),
                          preferred_element_type=jnp.float32
                          ).astype(jnp.int32)
        q_my = jnp.where((rank_my == kio) & m_my, 1.0, 0.0)

        px = lax.dot_general(
            q_my, x_ref[:, :], (((0,), (0,)), ((), ())),
            preferred_element_type=jnp.float32)
        sel = lax.dot_general(
            q_my, (idx % EXPERTS_PER_DEV).astype(jnp.float32),
            (((0,), (0,)), ((), ())),
            preferred_element_type=jnp.float32)
        y0 = jnp.dot(px, ew_ref[0, :, :], preferred_element_type=jnp.float32)
        y1 = jnp.dot(px, ew_ref[1, :, :], preferred_element_type=jnp.float32)
        packed = jnp.where(sel > 0.5, y1, y0)
        stage_ref[my] = packed.astype(jnp.bfloat16)

        pl.semaphore_wait(barrier_sem, N_DEV - 1)

        descs = []
        for j in range(1, N_DEV):
            r = (my + j) % N_DEV
            dsc = pltpu.make_async_remote_copy(
                src_ref=stage_ref.at[my],
                dst_ref=stage_ref.at[my],
                send_sem=ssems.at[j - 1],
                recv_sem=rsems.at[j - 1],
                device_id=(r,),
                device_id_type=pl.DeviceIdType.MESH,
            )
            dsc.start()
            descs.append(dsc)

        sdev = lax.broadcasted_iota(jnp.int32, (n, N_DEV), 1)
        m_all = (dev == sdev)
        ranks = jnp.dot(ltri, m_all.astype(jnp.float32),
                        preferred_element_type=jnp.float32
                        ).astype(jnp.int32)
        for s in range(N_DEV):
            q_s = jnp.where(
                (ranks[:, s:s + 1] == kio) & m_all[:, s:s + 1], 1.0, 0.0)
            u_ref[:, s * CAP:(s + 1) * CAP] = q_s.astype(jnp.bfloat16)

        for dsc in descs:
            dsc.wait()

        stacked = stage_ref[:, :, :].reshape(N_DEV * CAP, h)
        out_ref[:, :] = jnp.dot(u_ref[:, :], stacked,
                                preferred_element_type=jnp.float32)

    return pl.pallas_call(
        body,
        out_shape=jax.ShapeDtypeStruct((n, h), jnp.float32),
        in_specs=[
            pl.BlockSpec(memory_space=pltpu.VMEM),
            pl.BlockSpec(memory_space=pltpu.VMEM),
            pl.BlockSpec(memory_space=pltpu.VMEM),
        ],
        out_specs=pl.BlockSpec(memory_space=pltpu.VMEM),
        scratch_shapes=[
            pltpu.VMEM((N_DEV, CAP, h), jnp.bfloat16),
            pltpu.VMEM((n, N_DEV * CAP), jnp.bfloat16),
            pltpu.SemaphoreType.DMA((N_DEV - 1,)),
            pltpu.SemaphoreType.DMA((N_DEV - 1,)),
        ],
        compiler_params=pltpu.CompilerParams(collective_id=0),
    )(x, route_idx, expert_W)
